# baseline (speedup 1.0000x reference)
import jax
import jax.numpy as jnp
from jax import lax
from jax.experimental import pallas as pl
from jax.experimental.pallas import tpu as pltpu

NUM_CLASSES = 1000
ROWS_PER_BLOCK = 32
NBUF = 4


def _onehot_manual(x_ref, o_hbm, buf, sems):
    i = pl.program_id(0)
    slot = lax.rem(i, NBUF)

    @pl.when(i >= NBUF)
    def _():
        pltpu.make_async_copy(
            buf.at[slot],
            o_hbm.at[pl.ds((i - NBUF) * ROWS_PER_BLOCK, ROWS_PER_BLOCK)],
            sems.at[slot],
        ).wait()

    idx = x_ref[...]
    iota = lax.broadcasted_iota(
        jnp.int32, (ROWS_PER_BLOCK, x_ref.shape[1], NUM_CLASSES), 2
    )
    buf[slot] = (idx == iota).astype(jnp.float32)
    pltpu.make_async_copy(
        buf.at[slot],
        o_hbm.at[pl.ds(i * ROWS_PER_BLOCK, ROWS_PER_BLOCK)],
        sems.at[slot],
    ).start()

    ng = pl.num_programs(0)

    @pl.when(i == ng - 1)
    def _():
        for k in range(NBUF):
            step = ng - NBUF + k
            s = step % NBUF
            pltpu.make_async_copy(
                buf.at[s],
                o_hbm.at[pl.ds(step * ROWS_PER_BLOCK, ROWS_PER_BLOCK)],
                sems.at[s],
            ).wait()


def kernel(x):
    n, m = x.shape
    grid = n // ROWS_PER_BLOCK
    return pl.pallas_call(
        _onehot_manual,
        grid=(grid,),
        in_specs=[pl.BlockSpec((ROWS_PER_BLOCK, m, 1), lambda i: (i, 0, 0))],
        out_specs=pl.BlockSpec(memory_space=pl.ANY),
        out_shape=jax.ShapeDtypeStruct((n, m, NUM_CLASSES), jnp.float32),
        scratch_shapes=[
            pltpu.VMEM((NBUF, ROWS_PER_BLOCK, m, NUM_CLASSES), jnp.float32),
            pltpu.SemaphoreType.DMA((NBUF,)),
        ],
    )(x[:, :, None])
